# Initial kernel scaffold; baseline (speedup 1.0000x reference)
import jax
import jax.numpy as jnp
from jax.experimental import pallas as pl


def kernel(output, input, psedo_curve, step):
    raise NotImplementedError("write your pallas kernel here")


# speedup vs baseline: 136.9083x; 136.9083x over previous
import functools

import jax
import jax.numpy as jnp
from jax import lax
from jax.experimental import pallas as pl
from jax.experimental.pallas import tpu as pltpu
from jax.experimental.pallas import tpu_sc as plsc

B = 32
H = 512
W = 512
C = 3
NBINS = 255
LAMBDA_SMOOTH = 0.1

ROW_WORDS = W * C
ROWS_PER_IMG = H // 4
PIX = W // 4
CHUNK = 64
NCHUNK = ROWS_PER_IMG // CHUNK
HIST_PAD = 256

_mesh = plsc.VectorSubcoreMesh(core_axis_name="c", subcore_axis_name="s")


@functools.partial(
    pl.kernel,
    out_type=jax.ShapeDtypeStruct((B, HIST_PAD), jnp.float32),
    mesh=_mesh,
    scratch_types=[
        pltpu.VMEM((CHUNK,), jnp.int32),
        pltpu.VMEM((CHUNK, ROW_WORDS), jnp.float32),
        pltpu.VMEM((HIST_PAD,), jnp.float32),
        pltpu.SemaphoreType.DMA,
    ],
    compiler_params=pltpu.CompilerParams(
        use_tc_tiling_on_sc=False,
        needs_layout_passes=False,
    ),
)
def _sc_hist(x_hbm, out_hbm, idx_v, rows_v, hist_v, sem):
    c = lax.axis_index("c")
    s = lax.axis_index("s")
    wid = s * 2 + c
    lanes = lax.iota(jnp.int32, 16)
    ones16 = jnp.ones((16,), jnp.float32)
    zeros16 = jnp.zeros((16,), jnp.float32)

    for i in range(HIST_PAD // 16):
        hist_v[pl.ds(i * 16, 16)] = zeros16

    for chunk in range(NCHUNK):
        for j in range(CHUNK // 16):
            idx_v[pl.ds(j * 16, 16)] = wid * H + (chunk * CHUNK + j * 16 + lanes) * 4
        pltpu.async_copy(x_hbm.at[idx_v], rows_v, sem).wait()

        def row_body(r, carry):
            row_idx = jnp.full((16,), 0, jnp.int32) + r
            for g in range(PIX // 16):
                cols = (g * 16 + lanes) * 12
                acc = plsc.load_gather(rows_v, [row_idx, cols])
                acc = acc + plsc.load_gather(rows_v, [row_idx, cols + 1])
                acc = acc + plsc.load_gather(rows_v, [row_idx, cols + 2])
                mean = acc * jnp.float32(1.0 / 3.0)
                b_idx = (mean * jnp.float32(NBINS)).astype(jnp.int32)
                b_idx = jnp.minimum(jnp.maximum(b_idx, 0), NBINS - 1)
                plsc.addupdate_scatter(hist_v, [b_idx], ones16)
            return carry

        lax.fori_loop(0, CHUNK, row_body, 0)

    pltpu.sync_copy(hist_v, out_hbm.at[wid])


def _tc_loss_body(step_ref, hist_ref, out_ref, psedo_ref, loss_ref):
    hp = hist_ref[...]
    hist = jnp.sum(hp, axis=0, keepdims=True)
    total = jnp.sum(hist)
    ii = lax.broadcasted_iota(jnp.int32, (HIST_PAD, HIST_PAD), 0)
    jj = lax.broadcasted_iota(jnp.int32, (HIST_PAD, HIST_PAD), 1)
    tri = (ii <= jj).astype(jnp.float32)
    cdf = jnp.dot(hist, tri, preferred_element_type=jnp.float32)
    cdfn = cdf[:, :NBINS] / total

    output = out_ref[...]
    psedo = psedo_ref[...]
    curve_loss = jnp.mean((output - cdfn) ** 2)
    psedo_loss = jnp.mean((psedo - output) ** 2) + 0.01 * jnp.mean((psedo - cdfn) ** 2)
    smooth_loss = jnp.mean((output[:, 1:] - output[:, :-1]) ** 2)
    w = jnp.where(step_ref[0] >= 3000, jnp.float32(0.5), jnp.float32(1.0))
    total_loss = w * curve_loss + LAMBDA_SMOOTH * smooth_loss + 0.5 * psedo_loss
    loss_ref[...] = jnp.full((1, 1), total_loss, jnp.float32)


_tc_loss = pl.pallas_call(
    _tc_loss_body,
    out_shape=jax.ShapeDtypeStruct((1, 1), jnp.float32),
    in_specs=[
        pl.BlockSpec(memory_space=pltpu.SMEM),
        pl.BlockSpec(memory_space=pltpu.VMEM),
        pl.BlockSpec(memory_space=pltpu.VMEM),
        pl.BlockSpec(memory_space=pltpu.VMEM),
    ],
    out_specs=pl.BlockSpec(memory_space=pltpu.VMEM),
)


def kernel(output, input, psedo_curve, step):
    x = input.reshape(B * H, W * C)
    hist_parts = _sc_hist(x)
    step_arr = jnp.asarray(step, jnp.int32).reshape(1)
    loss = _tc_loss(step_arr, hist_parts, output, psedo_curve)
    return loss.reshape(())


# speedup vs baseline: 403.6746x; 2.9485x over previous
import functools

import jax
import jax.numpy as jnp
from jax import lax
from jax.experimental import pallas as pl
from jax.experimental.pallas import tpu as pltpu
from jax.experimental.pallas import tpu_sc as plsc

B = 32
H = 512
W = 512
C = 3
NBINS = 255
LAMBDA_SMOOTH = 0.1

ROWS_PER_IMG = H // 4
PIX = W // 4
CHUNK = 64
NCHUNK = ROWS_PER_IMG // CHUNK
HIST_PAD = 256

_mesh = plsc.VectorSubcoreMesh(core_axis_name="c", subcore_axis_name="s")


@functools.partial(
    pl.kernel,
    out_type=jax.ShapeDtypeStruct((B, HIST_PAD), jnp.float32),
    mesh=_mesh,
    scratch_types=[
        pltpu.VMEM((3 * CHUNK,), jnp.int32),
        pltpu.VMEM((3 * CHUNK, W), jnp.float32),
        pltpu.VMEM((HIST_PAD,), jnp.float32),
        pltpu.SemaphoreType.DMA,
    ],
    compiler_params=pltpu.CompilerParams(
        use_tc_tiling_on_sc=False,
        needs_layout_passes=False,
    ),
)
def _sc_hist(x_hbm, out_hbm, idx_v, rows_v, hist_v, sem):
    c = lax.axis_index("c")
    s = lax.axis_index("s")
    wid = s * 2 + c
    lanes = lax.iota(jnp.int32, 16)
    ones16 = jnp.ones((16,), jnp.float32)
    zeros16 = jnp.zeros((16,), jnp.float32)

    for i in range(HIST_PAD // 16):
        hist_v[pl.ds(i * 16, 16)] = zeros16

    for chunk in range(NCHUNK):
        for ch in range(3):
            for j in range(CHUNK // 16):
                idx_v[pl.ds(ch * CHUNK + j * 16, 16)] = (
                    (wid * 3 + ch) * H + (chunk * CHUNK + j * 16 + lanes) * 4
                )
        pltpu.async_copy(x_hbm.at[idx_v], rows_v, sem).wait()

        def row_body(r, carry):
            r0 = jnp.full((16,), 0, jnp.int32) + r
            for g in range(PIX // 16):
                cols = (g * 16 + lanes) * 4
                acc = plsc.load_gather(rows_v, [r0, cols])
                acc = acc + plsc.load_gather(rows_v, [r0 + CHUNK, cols])
                acc = acc + plsc.load_gather(rows_v, [r0 + 2 * CHUNK, cols])
                mean = acc * jnp.float32(1.0 / 3.0)
                b_idx = (mean * jnp.float32(NBINS)).astype(jnp.int32)
                b_idx = jnp.minimum(jnp.maximum(b_idx, 0), NBINS - 1)
                plsc.addupdate_scatter(hist_v, [b_idx], ones16)
            return carry

        lax.fori_loop(0, CHUNK, row_body, 0)

    pltpu.sync_copy(hist_v, out_hbm.at[wid])


def _tc_loss_body(step_ref, hist_ref, out_ref, psedo_ref, loss_ref):
    hp = hist_ref[...]
    hist = jnp.sum(hp, axis=0, keepdims=True)
    total = jnp.sum(hist)
    ii = lax.broadcasted_iota(jnp.int32, (HIST_PAD, HIST_PAD), 0)
    jj = lax.broadcasted_iota(jnp.int32, (HIST_PAD, HIST_PAD), 1)
    tri = (ii <= jj).astype(jnp.float32)
    cdf = jnp.dot(hist, tri, preferred_element_type=jnp.float32)
    cdfn = cdf[:, :NBINS] / total

    output = out_ref[...]
    psedo = psedo_ref[...]
    curve_loss = jnp.mean((output - cdfn) ** 2)
    psedo_loss = jnp.mean((psedo - output) ** 2) + 0.01 * jnp.mean((psedo - cdfn) ** 2)
    smooth_loss = jnp.mean((output[:, 1:] - output[:, :-1]) ** 2)
    w = jnp.where(step_ref[0] >= 3000, jnp.float32(0.5), jnp.float32(1.0))
    total_loss = w * curve_loss + LAMBDA_SMOOTH * smooth_loss + 0.5 * psedo_loss
    loss_ref[...] = jnp.full((1, 1), total_loss, jnp.float32)


_tc_loss = pl.pallas_call(
    _tc_loss_body,
    out_shape=jax.ShapeDtypeStruct((1, 1), jnp.float32),
    in_specs=[
        pl.BlockSpec(memory_space=pltpu.SMEM),
        pl.BlockSpec(memory_space=pltpu.VMEM),
        pl.BlockSpec(memory_space=pltpu.VMEM),
        pl.BlockSpec(memory_space=pltpu.VMEM),
    ],
    out_specs=pl.BlockSpec(memory_space=pltpu.VMEM),
)


def kernel(output, input, psedo_curve, step):
    x = jnp.transpose(input, (0, 3, 1, 2)).reshape(B * C * H, W)
    hist_parts = _sc_hist(x)
    step_arr = jnp.asarray(step, jnp.int32).reshape(1)
    loss = _tc_loss(step_arr, hist_parts, output, psedo_curve)
    return loss.reshape(())


# speedup vs baseline: 981.7837x; 2.4321x over previous
import functools

import jax
import jax.numpy as jnp
from jax import lax
from jax.experimental import pallas as pl
from jax.experimental.pallas import tpu as pltpu
from jax.experimental.pallas import tpu_sc as plsc

B = 32
H = 512
W = 512
C = 3
NBINS = 255
LAMBDA_SMOOTH = 0.1

HIST_PAD = 256
TBLK = H // 8
UBLK = W // 128
SEG_PER_CHUNK = 768
QPC = 256
NCHUNK = 2

_mesh = plsc.VectorSubcoreMesh(core_axis_name="c", subcore_axis_name="s")


@functools.partial(
    pl.kernel,
    out_type=jax.ShapeDtypeStruct((B, HIST_PAD), jnp.float32),
    mesh=_mesh,
    scratch_types=[
        pltpu.VMEM((SEG_PER_CHUNK,), jnp.int32),
        pltpu.VMEM((SEG_PER_CHUNK, 128), jnp.float32),
        pltpu.VMEM((HIST_PAD,), jnp.float32),
        pltpu.SemaphoreType.DMA,
    ],
    compiler_params=pltpu.CompilerParams(
        use_tc_tiling_on_sc=False,
        needs_layout_passes=False,
    ),
)
def _sc_hist(x_hbm, out_hbm, idx_v, segs_v, hist_v, sem):
    c = lax.axis_index("c")
    s = lax.axis_index("s")
    wid = s * 2 + c
    lanes = lax.iota(jnp.int32, 16)
    ones16 = jnp.ones((16,), jnp.float32)
    zeros16 = jnp.zeros((16,), jnp.float32)

    for i in range(HIST_PAD // 16):
        hist_v[pl.ds(i * 16, 16)] = zeros16

    for chunk in range(NCHUNK):
        for ch in range(3):
            for j in range(QPC // 16):
                q = j * 16 + lanes
                m = ((wid * 3 + ch) * 2048
                     + (chunk * (TBLK // 2) + (q >> 3)) * 32
                     + ((q >> 1) & 3) * 8
                     + (q & 1) * 4)
                idx_v[pl.ds(ch * QPC + j * 16, 16)] = m
        pltpu.async_copy(x_hbm.at[idx_v], segs_v, sem).wait()

        def seg_body(q, carry):
            r0 = jnp.full((16,), 0, jnp.int32) + q
            for g in range(2):
                cols = (g * 16 + lanes) * 4
                acc = plsc.load_gather(segs_v, [r0, cols])
                acc = acc + plsc.load_gather(segs_v, [r0 + QPC, cols])
                acc = acc + plsc.load_gather(segs_v, [r0 + 2 * QPC, cols])
                mean = acc * jnp.float32(1.0 / 3.0)
                b_idx = (mean * jnp.float32(NBINS)).astype(jnp.int32)
                b_idx = jnp.minimum(jnp.maximum(b_idx, 0), NBINS - 1)
                plsc.addupdate_scatter(hist_v, [b_idx], ones16)
            return carry

        lax.fori_loop(0, QPC, seg_body, 0)

    pltpu.sync_copy(hist_v, out_hbm.at[wid])


def _tc_loss_body(step_ref, hist_ref, out_ref, psedo_ref, loss_ref):
    hp = hist_ref[...]
    hist = jnp.sum(hp, axis=0, keepdims=True)
    total = jnp.sum(hist)
    ii = lax.broadcasted_iota(jnp.int32, (HIST_PAD, HIST_PAD), 0)
    jj = lax.broadcasted_iota(jnp.int32, (HIST_PAD, HIST_PAD), 1)
    tri = (ii <= jj).astype(jnp.float32)
    cdf = jnp.dot(hist, tri, preferred_element_type=jnp.float32)
    cdfn = cdf[:, :NBINS] / total

    output = out_ref[...]
    psedo = psedo_ref[...]
    curve_loss = jnp.mean((output - cdfn) ** 2)
    psedo_loss = jnp.mean((psedo - output) ** 2) + 0.01 * jnp.mean((psedo - cdfn) ** 2)
    smooth_loss = jnp.mean((output[:, 1:] - output[:, :-1]) ** 2)
    w = jnp.where(step_ref[0] >= 3000, jnp.float32(0.5), jnp.float32(1.0))
    total_loss = w * curve_loss + LAMBDA_SMOOTH * smooth_loss + 0.5 * psedo_loss
    loss_ref[...] = jnp.full((1, 1), total_loss, jnp.float32)


_tc_loss = pl.pallas_call(
    _tc_loss_body,
    out_shape=jax.ShapeDtypeStruct((1, 1), jnp.float32),
    in_specs=[
        pl.BlockSpec(memory_space=pltpu.SMEM),
        pl.BlockSpec(memory_space=pltpu.VMEM),
        pl.BlockSpec(memory_space=pltpu.VMEM),
        pl.BlockSpec(memory_space=pltpu.VMEM),
    ],
    out_specs=pl.BlockSpec(memory_space=pltpu.VMEM),
)


def kernel(output, input, psedo_curve, step):
    x = (jnp.transpose(input, (0, 3, 1, 2))
         .reshape(B * C, TBLK, 8, UBLK, 128)
         .transpose(0, 1, 3, 2, 4)
         .reshape(B * C * H * UBLK, 128))
    hist_parts = _sc_hist(x)
    step_arr = jnp.asarray(step, jnp.int32).reshape(1)
    loss = _tc_loss(step_arr, hist_parts, output, psedo_curve)
    return loss.reshape(())


# speedup vs baseline: 1060.3516x; 1.0800x over previous
import functools

import jax
import jax.numpy as jnp
from jax import lax
from jax.experimental import pallas as pl
from jax.experimental.pallas import tpu as pltpu
from jax.experimental.pallas import tpu_sc as plsc

B = 32
H = 512
W = 512
C = 3
NBINS = 255
LAMBDA_SMOOTH = 0.1

HIST_PAD = 256
TBLK = H // 8
UBLK = W // 128
QPC = 128
SEG_PER_CHUNK = 3 * QPC
NCHUNK = 4

_mesh = plsc.VectorSubcoreMesh(core_axis_name="c", subcore_axis_name="s")


@functools.partial(
    pl.kernel,
    out_type=jax.ShapeDtypeStruct((B, HIST_PAD), jnp.float32),
    mesh=_mesh,
    scratch_types=[
        pltpu.VMEM((SEG_PER_CHUNK,), jnp.int32),
        pltpu.VMEM((SEG_PER_CHUNK,), jnp.int32),
        pltpu.VMEM((SEG_PER_CHUNK, 128), jnp.float32),
        pltpu.VMEM((SEG_PER_CHUNK, 128), jnp.float32),
        pltpu.VMEM((HIST_PAD,), jnp.float32),
        pltpu.SemaphoreType.DMA,
        pltpu.SemaphoreType.DMA,
    ],
    compiler_params=pltpu.CompilerParams(
        use_tc_tiling_on_sc=False,
        needs_layout_passes=False,
    ),
)
def _sc_hist(x_hbm, out_hbm, idx_a, idx_b, segs_a, segs_b, hist_v, sem_a, sem_b):
    c = lax.axis_index("c")
    s = lax.axis_index("s")
    wid = s * 2 + c
    lanes = lax.iota(jnp.int32, 16)
    ones16 = jnp.ones((16,), jnp.float32)
    zeros16 = jnp.zeros((16,), jnp.float32)
    idx_bufs = [idx_a, idx_b]
    seg_bufs = [segs_a, segs_b]
    sems = [sem_a, sem_b]

    for i in range(HIST_PAD // 16):
        hist_v[pl.ds(i * 16, 16)] = zeros16

    def build_idx(chunk, idx_v):
        for ch in range(3):
            for j in range(QPC // 16):
                q = j * 16 + lanes
                m = ((wid * 3 + ch) * 2048
                     + (chunk * (TBLK // NCHUNK) + (q >> 3)) * 32
                     + ((q >> 1) & 3) * 8
                     + (q & 1) * 4)
                idx_v[pl.ds(ch * QPC + j * 16, 16)] = m

    def compute(segs_v):
        def seg_body(i, carry):
            for half in range(2):
                r0 = jnp.full((16,), 0, jnp.int32) + (i * 2 + half)
                for g in range(2):
                    cols = (g * 16 + lanes) * 4
                    acc = plsc.load_gather(segs_v, [r0, cols])
                    acc = acc + plsc.load_gather(segs_v, [r0 + QPC, cols])
                    acc = acc + plsc.load_gather(segs_v, [r0 + 2 * QPC, cols])
                    b_idx = (acc * jnp.float32(NBINS / 3.0)).astype(jnp.int32)
                    b_idx = jnp.minimum(jnp.maximum(b_idx, 0), NBINS - 1)
                    plsc.addupdate_scatter(hist_v, [b_idx], ones16)
            return carry

        lax.fori_loop(0, QPC // 2, seg_body, 0)

    build_idx(0, idx_bufs[0])
    copies = [pltpu.async_copy(x_hbm.at[idx_bufs[0]], seg_bufs[0], sems[0]), None]
    for chunk in range(NCHUNK):
        par = chunk % 2
        nxt = 1 - par
        if chunk + 1 < NCHUNK:
            build_idx(chunk + 1, idx_bufs[nxt])
            copies[nxt] = pltpu.async_copy(
                x_hbm.at[idx_bufs[nxt]], seg_bufs[nxt], sems[nxt])
        copies[par].wait()
        compute(seg_bufs[par])

    pltpu.sync_copy(hist_v, out_hbm.at[wid])


def _tc_loss_body(step_ref, hist_ref, out_ref, psedo_ref, loss_ref):
    hp = hist_ref[...]
    hist = jnp.sum(hp, axis=0, keepdims=True)
    total = jnp.sum(hist)
    ii = lax.broadcasted_iota(jnp.int32, (HIST_PAD, HIST_PAD), 0)
    jj = lax.broadcasted_iota(jnp.int32, (HIST_PAD, HIST_PAD), 1)
    tri = (ii <= jj).astype(jnp.float32)
    cdf = jnp.dot(hist, tri, preferred_element_type=jnp.float32)
    cdfn = cdf[:, :NBINS] / total

    output = out_ref[...]
    psedo = psedo_ref[...]
    curve_loss = jnp.mean((output - cdfn) ** 2)
    psedo_loss = jnp.mean((psedo - output) ** 2) + 0.01 * jnp.mean((psedo - cdfn) ** 2)
    smooth_loss = jnp.mean((output[:, 1:] - output[:, :-1]) ** 2)
    w = jnp.where(step_ref[0] >= 3000, jnp.float32(0.5), jnp.float32(1.0))
    total_loss = w * curve_loss + LAMBDA_SMOOTH * smooth_loss + 0.5 * psedo_loss
    loss_ref[...] = jnp.full((1, 1), total_loss, jnp.float32)


_tc_loss = pl.pallas_call(
    _tc_loss_body,
    out_shape=jax.ShapeDtypeStruct((1, 1), jnp.float32),
    in_specs=[
        pl.BlockSpec(memory_space=pltpu.SMEM),
        pl.BlockSpec(memory_space=pltpu.VMEM),
        pl.BlockSpec(memory_space=pltpu.VMEM),
        pl.BlockSpec(memory_space=pltpu.VMEM),
    ],
    out_specs=pl.BlockSpec(memory_space=pltpu.VMEM),
)


def kernel(output, input, psedo_curve, step):
    x = (jnp.transpose(input, (0, 3, 1, 2))
         .reshape(B * C, TBLK, 8, UBLK, 128)
         .transpose(0, 1, 3, 2, 4)
         .reshape(B * C * H * UBLK, 128))
    hist_parts = _sc_hist(x)
    step_arr = jnp.asarray(step, jnp.int32).reshape(1)
    loss = _tc_loss(step_arr, hist_parts, output, psedo_curve)
    return loss.reshape(())
